# baseline (speedup 1.0000x reference)
import functools

import jax
import jax.numpy as jnp
from jax import lax
from jax.experimental import pallas as pl
from jax.experimental.pallas import tpu as pltpu
from jax.experimental.pallas import tpu_sc as plsc

N = 10000
E = 320000
D = 128
DW = 128

NC = 2
NS = 16
NW = NC * NS
EPW = E // NW
DWP = DW // 2
CHUNK = 80
NCHUNK = EPW // CHUNK
GROUPS = CHUNK // 16


def _qk_body(x_ref, wqt_ref, bq_ref, wkt_ref, bk_ref, q_ref, k_ref):
    x = x_ref[...]
    inv_dk = 1.0 / (DW ** 0.5)
    q = jax.lax.dot_general(x, wqt_ref[...], (((1,), (0,)), ((), ())),
                            preferred_element_type=jnp.float32,
                            precision=jax.lax.Precision.HIGHEST)
    k = jax.lax.dot_general(x, wkt_ref[...], (((1,), (0,)), ((), ())),
                            preferred_element_type=jnp.float32,
                            precision=jax.lax.Precision.HIGHEST)
    q_ref[...] = ((q + bq_ref[...]) * inv_dk).astype(jnp.bfloat16)
    k_ref[...] = (k + bk_ref[...]).astype(jnp.bfloat16)


def _tc_qk(inputs, WqT, bq2, WkT, bk2):
    return pl.pallas_call(
        _qk_body,
        out_shape=(
            jax.ShapeDtypeStruct((N, DW), jnp.bfloat16),
            jax.ShapeDtypeStruct((N, DW), jnp.bfloat16),
        ),
    )(inputs, WqT, bq2, WkT, bk2)


def _sc_scores_body(q_hbm, k_hbm, src_hbm, dst_hbm, out_hbm,
                    src_v, dst_v, qr0, kr0, qr1, kr1, pbuf, out_v,
                    sem_q0, sem_k0, sem_q1, sem_k1):
    wid = lax.axis_index("s") * NC + lax.axis_index("c")
    base = wid * EPW
    pltpu.sync_copy(src_hbm.at[pl.ds(base, EPW)], src_v)
    pltpu.sync_copy(dst_hbm.at[pl.ds(base, EPW)], dst_v)

    lanes16 = lax.iota(jnp.int32, 16) * 16
    bufs = ((qr0, kr0, sem_q0, sem_k0), (qr1, kr1, sem_q1, sem_k1))

    def issue(c, b):
        qr, kr, sem_q, sem_k = bufs[b]
        cb = c * CHUNK
        pltpu.async_copy(q_hbm.at[src_v.at[pl.ds(cb, CHUNK)]], qr, sem_q)
        pltpu.async_copy(k_hbm.at[dst_v.at[pl.ds(cb, CHUNK)]], kr, sem_k)

    def wait(b):
        qr, kr, sem_q, sem_k = bufs[b]
        pltpu.make_async_copy(q_hbm.at[src_v.at[pl.ds(0, CHUNK)]], qr, sem_q).wait()
        pltpu.make_async_copy(k_hbm.at[dst_v.at[pl.ds(0, CHUNK)]], kr, sem_k).wait()

    def compute(c, b):
        qr, kr, _, _ = bufs[b]
        cb = c * CHUNK

        def group_body(g, carry2):
            eb = g * 16
            for l in range(16):
                e = eb + l
                acc = jnp.zeros((16,), jnp.float32)
                for j in range(DWP // 16):
                    qb = plsc.bitcast(qr[e, pl.ds(j * 16, 16)], jnp.bfloat16)
                    kb = plsc.bitcast(kr[e, pl.ds(j * 16, 16)], jnp.bfloat16)
                    qa, qc = plsc.unpack(qb, format=plsc.PackFormat.INTERLEAVED)
                    ka, kc = plsc.unpack(kb, format=plsc.PackFormat.INTERLEAVED)
                    acc = acc + qa * ka + qc * kc
                pbuf[pl.ds(l * 16, 16)] = acc
            o = jnp.zeros((16,), jnp.float32)
            for c2 in range(16):
                o = o + plsc.load_gather(pbuf, [lanes16 + c2])
            out_v[pl.ds(cb + eb, 16)] = o
            return carry2

        lax.fori_loop(0, GROUPS, group_body, 0)

    issue(0, 0)

    def pair_body(i, carry):
        c = 2 * i
        wait(0)
        issue(c + 1, 1)
        compute(c, 0)
        wait(1)
        issue(c + 2, 0)
        compute(c + 1, 1)
        return carry

    lax.fori_loop(0, (NCHUNK - 1) // 2, pair_body, 0)
    wait(0)
    compute(NCHUNK - 1, 0)
    pltpu.sync_copy(out_v, out_hbm.at[pl.ds(base, EPW)])


_sc_scores = functools.partial(
    pl.kernel,
    mesh=plsc.VectorSubcoreMesh(core_axis_name="c", subcore_axis_name="s"),
    out_type=jax.ShapeDtypeStruct((E,), jnp.float32),
    compiler_params=pltpu.CompilerParams(
        needs_layout_passes=False, use_tc_tiling_on_sc=False),
    scratch_types=[
        pltpu.VMEM((EPW,), jnp.int32),
        pltpu.VMEM((EPW,), jnp.int32),
        pltpu.VMEM((CHUNK, DWP), jnp.int32),
        pltpu.VMEM((CHUNK, DWP), jnp.int32),
        pltpu.VMEM((CHUNK, DWP), jnp.int32),
        pltpu.VMEM((CHUNK, DWP), jnp.int32),
        pltpu.VMEM((256,), jnp.float32),
        pltpu.VMEM((EPW,), jnp.float32),
        pltpu.SemaphoreType.DMA,
        pltpu.SemaphoreType.DMA,
        pltpu.SemaphoreType.DMA,
        pltpu.SemaphoreType.DMA,
    ],
)(_sc_scores_body)


def kernel(inputs, sparse_adj_indices, Wq, bq, Wk, bk):
    q, k = _tc_qk(inputs, Wq.T, bq.reshape(1, DW), Wk.T, bk.reshape(1, DW))
    qp = jax.lax.bitcast_convert_type(q.reshape(N, DWP, 2), jnp.int32)
    kp = jax.lax.bitcast_convert_type(k.reshape(N, DWP, 2), jnp.int32)
    src = sparse_adj_indices[0]
    dst = sparse_adj_indices[1]
    return _sc_scores(qp, kp, src, dst)


# speedup vs baseline: 1.3076x; 1.3076x over previous
import functools

import jax
import jax.numpy as jnp
from jax import lax
from jax.experimental import pallas as pl
from jax.experimental.pallas import tpu as pltpu
from jax.experimental.pallas import tpu_sc as plsc

N = 10000
E = 320000
D = 128
DW = 128

NC = 2
NS = 16
NW = NC * NS
EPW = E // NW
DWP = DW // 2
CHUNK = 80
NCHUNK = EPW // CHUNK
GROUPS = CHUNK // 16


def _round_bf16_bits(x):
    u = jax.lax.bitcast_convert_type(x, jnp.uint32)
    return (u + jnp.uint32(0x7FFF) + ((u >> 16) & jnp.uint32(1))) & jnp.uint32(0xFFFF0000)


def _pack_table(x):
    lo = _round_bf16_bits(x[:, :DWP]) >> 16
    hi = _round_bf16_bits(x[:, DWP:])
    return jax.lax.bitcast_convert_type(lo | hi, jnp.int32)


def _qk_body(x_ref, wqt_ref, bq_ref, wkt_ref, bk_ref, q_ref, k_ref):
    x = x_ref[...]
    inv_dk = 1.0 / (DW ** 0.5)
    q = jax.lax.dot_general(x, wqt_ref[...], (((1,), (0,)), ((), ())),
                            preferred_element_type=jnp.float32,
                            precision=jax.lax.Precision.HIGHEST)
    k = jax.lax.dot_general(x, wkt_ref[...], (((1,), (0,)), ((), ())),
                            preferred_element_type=jnp.float32,
                            precision=jax.lax.Precision.HIGHEST)
    q_ref[...] = _pack_table((q + bq_ref[...]) * inv_dk)
    k_ref[...] = _pack_table(k + bk_ref[...])


def _tc_qk(inputs, WqT, bq2, WkT, bk2):
    return pl.pallas_call(
        _qk_body,
        out_shape=(
            jax.ShapeDtypeStruct((N, DWP), jnp.int32),
            jax.ShapeDtypeStruct((N, DWP), jnp.int32),
        ),
    )(inputs, WqT, bq2, WkT, bk2)


def _sc_scores_body(q_hbm, k_hbm, src_hbm, dst_hbm, out_hbm,
                    src_v, dst_v, qr0, kr0, qr1, kr1, pbuf, out_v,
                    sem_q0, sem_k0, sem_q1, sem_k1):
    wid = lax.axis_index("s") * NC + lax.axis_index("c")
    base = wid * EPW
    pltpu.sync_copy(src_hbm.at[pl.ds(base, EPW)], src_v)
    pltpu.sync_copy(dst_hbm.at[pl.ds(base, EPW)], dst_v)

    lanes16 = lax.iota(jnp.int32, 16) * 16
    bufs = ((qr0, kr0, sem_q0, sem_k0), (qr1, kr1, sem_q1, sem_k1))

    def issue(c, b):
        qr, kr, sem_q, sem_k = bufs[b]
        cb = c * CHUNK
        pltpu.async_copy(q_hbm.at[src_v.at[pl.ds(cb, CHUNK)]], qr, sem_q)
        pltpu.async_copy(k_hbm.at[dst_v.at[pl.ds(cb, CHUNK)]], kr, sem_k)

    def wait(b):
        qr, kr, sem_q, sem_k = bufs[b]
        pltpu.make_async_copy(q_hbm.at[src_v.at[pl.ds(0, CHUNK)]], qr, sem_q).wait()
        pltpu.make_async_copy(k_hbm.at[dst_v.at[pl.ds(0, CHUNK)]], kr, sem_k).wait()

    def compute(c, b):
        qr, kr, _, _ = bufs[b]
        cb = c * CHUNK

        def group_body(g, carry2):
            eb = g * 16
            for l in range(16):
                e = eb + l
                hmask = jnp.full((16,), -0x10000, jnp.int32)
                acc0 = jnp.zeros((16,), jnp.float32)
                acc1 = jnp.zeros((16,), jnp.float32)
                for j in range(DWP // 16):
                    qw = qr[e, pl.ds(j * 16, 16)]
                    kw = kr[e, pl.ds(j * 16, 16)]
                    qlo = plsc.bitcast(qw << 16, jnp.float32)
                    klo = plsc.bitcast(kw << 16, jnp.float32)
                    qhi = plsc.bitcast(qw & hmask, jnp.float32)
                    khi = plsc.bitcast(kw & hmask, jnp.float32)
                    acc0 = acc0 + qlo * klo
                    acc1 = acc1 + qhi * khi
                pbuf[pl.ds(l * 16, 16)] = acc0 + acc1
            o = jnp.zeros((16,), jnp.float32)
            for c2 in range(16):
                o = o + plsc.load_gather(pbuf, [lanes16 + c2])
            out_v[pl.ds(cb + eb, 16)] = o
            return carry2

        lax.fori_loop(0, GROUPS, group_body, 0)

    issue(0, 0)

    def pair_body(i, carry):
        c = 2 * i
        wait(0)
        issue(c + 1, 1)
        compute(c, 0)
        wait(1)
        issue(c + 2, 0)
        compute(c + 1, 1)
        return carry

    lax.fori_loop(0, (NCHUNK - 1) // 2, pair_body, 0)
    wait(0)
    compute(NCHUNK - 1, 0)
    pltpu.sync_copy(out_v, out_hbm.at[pl.ds(base, EPW)])


_sc_scores = functools.partial(
    pl.kernel,
    mesh=plsc.VectorSubcoreMesh(core_axis_name="c", subcore_axis_name="s"),
    out_type=jax.ShapeDtypeStruct((E,), jnp.float32),
    compiler_params=pltpu.CompilerParams(
        needs_layout_passes=False, use_tc_tiling_on_sc=False),
    scratch_types=[
        pltpu.VMEM((EPW,), jnp.int32),
        pltpu.VMEM((EPW,), jnp.int32),
        pltpu.VMEM((CHUNK, DWP), jnp.int32),
        pltpu.VMEM((CHUNK, DWP), jnp.int32),
        pltpu.VMEM((CHUNK, DWP), jnp.int32),
        pltpu.VMEM((CHUNK, DWP), jnp.int32),
        pltpu.VMEM((256,), jnp.float32),
        pltpu.VMEM((EPW,), jnp.float32),
        pltpu.SemaphoreType.DMA,
        pltpu.SemaphoreType.DMA,
        pltpu.SemaphoreType.DMA,
        pltpu.SemaphoreType.DMA,
    ],
)(_sc_scores_body)


def kernel(inputs, sparse_adj_indices, Wq, bq, Wk, bk):
    qp, kp = _tc_qk(inputs, Wq.T, bq.reshape(1, DW), Wk.T, bk.reshape(1, DW))
    src = sparse_adj_indices[0]
    dst = sparse_adj_indices[1]
    return _sc_scores(qp, kp, src, dst)
